# baseline (speedup 1.0000x reference)
import functools

import jax
import jax.numpy as jnp
from jax import lax
from jax.experimental import pallas as pl
from jax.experimental.pallas import tpu as pltpu
from jax.experimental.pallas import tpu_sc as plsc

H = 8
NEG_SLOPE = 0.2
NUM_GRAPHS = 64
NSC = 2
NTEC = 16
ROWE = 128


def _lrelu(t):
    return jnp.where(t >= 0, t, NEG_SLOPE * t)


def _elu(t):
    return jnp.where(t > 0, t, jnp.exp(t) - 1.0)


def _attn_body(x_ref, ws_ref, wd_ref, as_ref, ad_ref, gmax_ref, acc_ref):
    i = pl.program_id(0)
    a_s = jnp.dot(x_ref[...], ws_ref[...], preferred_element_type=jnp.float32)
    a_d = jnp.dot(x_ref[...], wd_ref[...], preferred_element_type=jnp.float32)
    as_ref[...] = a_s
    ad_ref[...] = a_d
    bmax = jnp.max(a_s, axis=0, keepdims=True)

    @pl.when(i == 0)
    def _():
        acc_ref[...] = bmax

    @pl.when(i > 0)
    def _():
        acc_ref[...] = jnp.maximum(acc_ref[...], bmax)

    @pl.when(i == pl.num_programs(0) - 1)
    def _():
        gmax_ref[...] = acc_ref[...]


def _attn_scalars(x, ws, wd, nb):
    n, din = x.shape
    grid = (n // nb,)
    return pl.pallas_call(
        _attn_body,
        grid=grid,
        in_specs=[
            pl.BlockSpec((nb, din), lambda i: (i, 0)),
            pl.BlockSpec((din, H), lambda i: (0, 0)),
            pl.BlockSpec((din, H), lambda i: (0, 0)),
        ],
        out_specs=[
            pl.BlockSpec((nb, H), lambda i: (i, 0)),
            pl.BlockSpec((nb, H), lambda i: (i, 0)),
            pl.BlockSpec((1, H), lambda i: (0, 0)),
        ],
        out_shape=[
            jax.ShapeDtypeStruct((n, H), jnp.float32),
            jax.ShapeDtypeStruct((n, H), jnp.float32),
            jax.ShapeDtypeStruct((1, H), jnp.float32),
        ],
        scratch_shapes=[pltpu.VMEM((1, H), jnp.float32)],
    )(x, ws, wd)


def _mm_body(x_ref, w_ref, out_ref):
    out_ref[...] = jnp.dot(x_ref[...], w_ref[0],
                           preferred_element_type=jnp.float32)


def _matmul_heads(x, w_heads, nb):
    n, k = x.shape
    nh, _, c = w_heads.shape
    nblk = n // nb
    return pl.pallas_call(
        _mm_body,
        grid=(nblk, nh),
        in_specs=[
            pl.BlockSpec((nb, k), lambda i, h: (i, 0)),
            pl.BlockSpec((1, k, c), lambda i, h: (h, 0, 0)),
        ],
        out_specs=pl.BlockSpec((nb, c), lambda i, h: (h * nblk + i, 0)),
        out_shape=jax.ShapeDtypeStruct((nh * n, c), jnp.float32),
    )(x, w_heads)


def _norm_body(c, sp, agg_ref, den_ref, b_ref, out_ref):
    den = den_ref[0] + den_ref[1]
    cw = c // sp
    for h in range(H):
        d = jnp.clip(den[:, h:h + 1], 1e-16, None)
        for t in range(sp):
            v = agg_ref[h, t] / d + b_ref[0, h * c + t * cw:h * c + (t + 1) * cw]
            out_ref[:, h * c + t * cw:h * c + (t + 1) * cw] = _elu(v)


def _norm(agg, den, bias, nb):
    _, sp, n, cw = agg.shape
    c = sp * cw
    return pl.pallas_call(
        functools.partial(_norm_body, c, sp),
        grid=(n // nb,),
        in_specs=[
            pl.BlockSpec((H, sp, nb, cw), lambda i: (0, 0, i, 0)),
            pl.BlockSpec((2, nb, H), lambda i: (0, i, 0)),
            pl.BlockSpec((1, H * c), lambda i: (0, 0)),
        ],
        out_specs=pl.BlockSpec((nb, H * c), lambda i: (i, 0)),
        out_shape=jax.ShapeDtypeStruct((n, H * c), jnp.float32),
    )(agg, den, bias)


def _pool_body(z_ref, b_ref, w1_ref, b1_ref, w2_ref, b2_ref, out_ref):
    n = z_ref.shape[0]
    oh = (b_ref[...] == lax.broadcasted_iota(jnp.int32, (NUM_GRAPHS, n), 0))
    oh = oh.astype(jnp.float32)
    s = jnp.dot(oh, z_ref[...], preferred_element_type=jnp.float32)
    cnt = jnp.sum(oh, axis=1, keepdims=True)
    g = s / jnp.clip(cnt, 1.0, None)
    t = _elu(jnp.dot(g, w1_ref[...], preferred_element_type=jnp.float32)
             + b1_ref[...])
    out_ref[...] = jnp.dot(t, w2_ref[...],
                           preferred_element_type=jnp.float32) + b2_ref[...]


def _pool_mlp(z, batch2d, fc1_w, fc1_b, fc2_w, fc2_b):
    n, d = z.shape
    dh, do = fc1_w.shape[1], fc2_w.shape[1]
    return pl.pallas_call(
        _pool_body,
        out_shape=jax.ShapeDtypeStruct((NUM_GRAPHS, do), jnp.float32),
    )(z, batch2d, fc1_w, fc1_b.reshape(1, dh), fc2_w, fc2_b.reshape(1, do))


def _edge_sc(src2d, dst2d, ts, td, gmax, zeros8):
    er = src2d.shape[0]
    n = ts.shape[0]
    nrows = n // NTEC
    base = er // (NSC * NTEC)
    rem = er - base * NSC * NTEC
    nbody = base // 2
    assert base == 2 * nbody

    mesh = plsc.VectorSubcoreMesh(core_axis_name="c", subcore_axis_name="s")

    @functools.partial(
        pl.kernel,
        mesh=mesh,
        compiler_params=pltpu.CompilerParams(use_tc_tiling_on_sc=False, needs_layout_passes=False),
        out_type=[
            jax.ShapeDtypeStruct((er, ROWE, H), jnp.float32),
            jax.ShapeDtypeStruct((NSC, n, H), jnp.float32),
        ],
        scratch_types=[
            pltpu.VMEM((2, ROWE), jnp.int32),
            pltpu.VMEM((2, ROWE), jnp.int32),
            pltpu.VMEM((2, ROWE), jnp.int32),
            pltpu.VMEM((2, ROWE, H), jnp.float32),
            pltpu.VMEM((2, ROWE, H), jnp.float32),
            pltpu.VMEM((2, ROWE, H), jnp.float32),
            pltpu.VMEM((16,), jnp.float32),
            pltpu.VMEM_SHARED((n, H), jnp.float32),
            pltpu.SemaphoreType.DMA,
            pltpu.SemaphoreType.DMA,
            pltpu.SemaphoreType.DMA,
            pltpu.SemaphoreType.DMA,
            pltpu.SemaphoreType.DMA,
            pltpu.SemaphoreType.DMA,
            pltpu.SemaphoreType.DMA,
            pltpu.SemaphoreType.DMA,
        ],
    )
    def k(src_h, dst_h, ts_h, td_h, gmax_h, z8_h, ex_h, den_h,
          srcv, dstv, dsts, srows, drows, exrows, gbuf, denacc,
          semi0, semi1, semr0, semr1, semw0, semw1, semd0, semd1):
        c = lax.axis_index("c")
        s = lax.axis_index("s")
        wid = c * NTEC + s
        semi = [semi0, semi1]
        semr = [semr0, semr1]
        semw = [semw0, semw1]
        semd = [semd0, semd1]
        pltpu.sync_copy(z8_h.at[pl.ds(s * nrows, nrows), :],
                        denacc.at[pl.ds(s * nrows, nrows), :])
        pltpu.sync_copy(gmax_h, gbuf)
        plsc.subcore_barrier()

        lane = lax.broadcasted_iota(jnp.int32, (16,), 0)
        ci = lane & 7
        gv = plsc.load_gather(gbuf, [ci])
        start = wid * base + jnp.minimum(wid, rem)
        extra = wid < rem

        def fire_idx(k_, r):
            pltpu.async_copy(src_h.at[r], srcv.at[k_], semi[k_])
            pltpu.async_copy(dst_h.at[r], dstv.at[k_], semi[k_])

        def wait_idx(k_):
            pltpu.make_async_copy(src_h.at[0], srcv.at[k_], semi[k_]).wait()
            pltpu.make_async_copy(dst_h.at[0], dstv.at[k_], semi[k_]).wait()

        def prep_fire_rows(k_):
            def sb(i, _):
                dsts[k_, pl.ds(i * 16, 16)] = dstv[k_, pl.ds(i * 16, 16)]
                return 0
            lax.fori_loop(0, ROWE // 16, sb, 0)
            pltpu.async_copy(ts_h.at[srcv.at[k_]], srows.at[k_], semr[k_])
            pltpu.async_copy(td_h.at[dstv.at[k_]], drows.at[k_], semr[k_])

        def wait_rows(k_):
            pltpu.make_async_copy(ts_h.at[srcv.at[k_]], srows.at[k_],
                                  semr[k_]).wait()
            pltpu.make_async_copy(td_h.at[dstv.at[k_]], drows.at[k_],
                                  semr[k_]).wait()

        def compute(k_):
            ksp = jnp.full((16,), k_, jnp.int32)

            def body(i, _):
                ri = i * 2 + (lane >> 3)
                sv = plsc.load_gather(srows, [ksp, ri, ci])
                dv = plsc.load_gather(drows, [ksp, ri, ci])
                ex = jnp.exp(_lrelu(sv + dv) - _lrelu(gv + dv))
                plsc.store_scatter(exrows, [ksp, ri, ci], ex)
                return 0

            lax.fori_loop(0, ROWE * H // 16, body, 0)

        def fire_writes(k_, r):
            pltpu.async_copy(exrows.at[k_], ex_h.at[r], semw[k_])
            pltpu.async_copy(exrows.at[k_], denacc.at[dsts.at[k_]],
                             semd[k_], add=True)

        def wait_writes(k_):
            pltpu.make_async_copy(exrows.at[k_], ex_h.at[0], semw[k_]).wait()
            pltpu.make_async_copy(exrows.at[k_], denacc.at[dsts.at[k_]],
                                  semd[k_]).wait()

        fire_idx(0, start)

        def blk(g, _):
            wait_idx(0)

            @pl.when(g > 0)
            def _():
                wait_writes(0)

            prep_fire_rows(0)

            @pl.when(g > 0)
            def _():
                wait_rows(1)
                compute(1)
                fire_writes(1, start + 2 * g - 1)

            fire_idx(1, start + 2 * g + 1)
            wait_idx(1)

            @pl.when(g > 0)
            def _():
                wait_writes(1)

            prep_fire_rows(1)
            wait_rows(0)
            compute(0)
            fire_writes(0, start + 2 * g)

            @pl.when(g < nbody - 1)
            def _():
                fire_idx(0, start + 2 * g + 2)

            return 0

        lax.fori_loop(0, nbody, blk, 0)
        wait_rows(1)
        compute(1)
        pltpu.sync_copy(exrows.at[1], ex_h.at[start + base - 1])
        pltpu.sync_copy(exrows.at[1], denacc.at[dsts.at[1]], add=True)
        wait_writes(0)

        @pl.when(extra)
        def _():
            r = start + base
            pltpu.sync_copy(src_h.at[r], srcv.at[0])
            pltpu.sync_copy(dst_h.at[r], dstv.at[0])

            def sb(i, _):
                dsts[0, pl.ds(i * 16, 16)] = dstv[0, pl.ds(i * 16, 16)]
                return 0
            lax.fori_loop(0, ROWE // 16, sb, 0)
            pltpu.async_copy(ts_h.at[srcv.at[0]], srows.at[0], semr0)
            pltpu.async_copy(td_h.at[dstv.at[0]], drows.at[0], semr0)
            wait_rows(0)
            compute(0)
            pltpu.sync_copy(exrows.at[0], ex_h.at[r])
            pltpu.sync_copy(exrows.at[0], denacc.at[dsts.at[0]], add=True)

        plsc.subcore_barrier()
        pltpu.sync_copy(denacc.at[pl.ds(s * nrows, nrows), :],
                        den_h.at[c, pl.ds(s * nrows, nrows), :])

    return k(src2d, dst2d, ts, td, gmax, zeros8)


def _agg_sc(src2d, dst2d, ex3, hfeat, zeros_c, qblk, vh, shift):
    er = src2d.shape[0]
    n = zeros_c.shape[0]
    cw = zeros_c.shape[1]
    nrows = n // NTEC
    hpc = vh // NSC
    base = er // NTEC
    nblk = base // qblk
    nbody = nblk // 3
    rem = er - base * NTEC
    assert base == nblk * qblk and nblk == 3 * nbody

    mesh = plsc.VectorSubcoreMesh(core_axis_name="c", subcore_axis_name="s")

    @functools.partial(
        pl.kernel,
        mesh=mesh,
        compiler_params=pltpu.CompilerParams(use_tc_tiling_on_sc=False, needs_layout_passes=False),
        out_type=jax.ShapeDtypeStruct((vh * n, cw), jnp.float32),
        scratch_types=[
            pltpu.VMEM((3, qblk, ROWE), jnp.int32),
            pltpu.VMEM((3, qblk, ROWE), jnp.int32),
            pltpu.VMEM((3, qblk, ROWE), jnp.int32),
            pltpu.VMEM((3, qblk, ROWE), jnp.int32),
            pltpu.VMEM((3, qblk, ROWE, H), jnp.float32),
            pltpu.VMEM((3, qblk, ROWE, cw), jnp.float32),
            pltpu.VMEM_SHARED((n, cw), jnp.float32),
            pltpu.SemaphoreType.DMA,
            pltpu.SemaphoreType.DMA,
            pltpu.SemaphoreType.DMA,
            pltpu.SemaphoreType.DMA,
            pltpu.SemaphoreType.DMA,
            pltpu.SemaphoreType.DMA,
            pltpu.SemaphoreType.DMA,
            pltpu.SemaphoreType.DMA,
            pltpu.SemaphoreType.DMA,
        ],
    )
    def k(src_h, dst_h, ex_h, hf_h, zc_h, out_h,
          srcv, dstv, dsts, gidx, exv, gbuf, uacc,
          semi0, semi1, semi2, semg0, semg1, semg2, sems0, sems1, sems2):
        c = lax.axis_index("c")
        s = lax.axis_index("s")
        start = s * base + jnp.minimum(s, rem)
        extra = s < rem
        semi = [semi0, semi1, semi2]
        semg = [semg0, semg1, semg2]
        sems = [sems0, sems1, sems2]

        def fire_idx(k_, row0, q=qblk):
            pltpu.async_copy(src_h.at[pl.ds(row0, q)], srcv.at[k_, pl.ds(0, q)], semi[k_])
            pltpu.async_copy(dst_h.at[pl.ds(row0, q)], dstv.at[k_, pl.ds(0, q)], semi[k_])
            pltpu.async_copy(ex_h.at[pl.ds(row0, q)], exv.at[k_, pl.ds(0, q)], semi[k_])

        def wait_idx(k_, q=qblk):
            pltpu.make_async_copy(src_h.at[pl.ds(0, q)], srcv.at[k_, pl.ds(0, q)], semi[k_]).wait()
            pltpu.make_async_copy(dst_h.at[pl.ds(0, q)], dstv.at[k_, pl.ds(0, q)], semi[k_]).wait()
            pltpu.make_async_copy(ex_h.at[pl.ds(0, q)], exv.at[k_, pl.ds(0, q)], semi[k_]).wait()

        def fire_gather(k_, v, q=qblk):
            def rbody(i, _):
                qq = i >> 3
                ii = i & 7
                gidx[k_, qq, pl.ds(ii * 16, 16)] = (
                    srcv[k_, qq, pl.ds(ii * 16, 16)] + v * n)
                dsts[k_, qq, pl.ds(ii * 16, 16)] = dstv[k_, qq, pl.ds(ii * 16, 16)]
                return 0
            lax.fori_loop(0, q * 8, rbody, 0)
            for qq in range(q):
                pltpu.async_copy(hf_h.at[gidx.at[k_, qq]], gbuf.at[k_, qq],
                                 semg[k_])

        def wait_gather(k_, q=qblk):
            for qq in range(q):
                pltpu.make_async_copy(hf_h.at[gidx.at[k_, qq]],
                                      gbuf.at[k_, qq], semg[k_]).wait()

        def scale_scatter(k_, v, q=qblk):
            hsp = jnp.full((16,), 0, jnp.int32) + (v >> shift)
            ksp = jnp.full((16,), k_, jnp.int32)

            def row(r2, _):
                for u in range(2):
                    r = r2 * 2 + u
                    qq = r >> 7
                    rr = r & 127
                    rsp = jnp.full((16,), 0, jnp.int32) + rr
                    qsp = jnp.full((16,), 0, jnp.int32) + qq
                    exb = plsc.load_gather(exv, [ksp, qsp, rsp, hsp])
                    for cc in range(cw // 16):
                        gbuf[k_, qq, rr, pl.ds(cc * 16, 16)] = (
                            gbuf[k_, qq, rr, pl.ds(cc * 16, 16)] * exb)
                return 0

            lax.fori_loop(0, q * ROWE // 2, row, 0)
            for qq in range(q):
                pltpu.async_copy(gbuf.at[k_, qq], uacc.at[dsts.at[k_, qq]],
                                 sems[k_], add=True)

        def wait_scatter(k_, q=qblk):
            for qq in range(q):
                pltpu.make_async_copy(gbuf.at[k_, qq],
                                      uacc.at[dsts.at[k_, qq]],
                                      sems[k_]).wait()

        for j in range(hpc):
            v = c * hpc + j
            pltpu.sync_copy(zc_h.at[pl.ds(s * nrows, nrows), :],
                            uacc.at[pl.ds(s * nrows, nrows), :])
            plsc.subcore_barrier()

            fire_idx(0, start)
            fire_idx(1, start + qblk)
            fire_idx(2, start + 2 * qblk)

            def blk(g, _):
                wait_idx(0)

                @pl.when(g > 0)
                def _():
                    wait_scatter(0)

                fire_gather(0, v)

                @pl.when(g > 0)
                def _():
                    wait_gather(2)
                    scale_scatter(2, v)
                    fire_idx(2, start + (3 * g + 2) * qblk)

                wait_idx(1)

                @pl.when(g > 0)
                def _():
                    wait_scatter(1)

                fire_gather(1, v)
                wait_gather(0)
                scale_scatter(0, v)

                @pl.when(g < nbody - 1)
                def _():
                    fire_idx(0, start + (3 * g + 3) * qblk)

                wait_idx(2)

                @pl.when(g > 0)
                def _():
                    wait_scatter(2)

                fire_gather(2, v)
                wait_gather(1)
                scale_scatter(1, v)

                @pl.when(g < nbody - 1)
                def _():
                    fire_idx(1, start + (3 * g + 4) * qblk)

                return 0

            lax.fori_loop(0, nbody, blk, 0)
            wait_gather(2)
            scale_scatter(2, v)
            wait_scatter(0)
            wait_scatter(1)
            wait_scatter(2)

            @pl.when(extra)
            def _():
                fire_idx(0, start + base, 1)
                wait_idx(0, 1)
                fire_gather(0, v, 1)
                wait_gather(0, 1)
                scale_scatter(0, v, 1)
                wait_scatter(0, 1)

            plsc.subcore_barrier()
            pltpu.sync_copy(uacc.at[pl.ds(s * nrows, nrows), :],
                            out_h.at[pl.ds(v * n + s * nrows, nrows), :])
            plsc.subcore_barrier()

    return k(src2d, dst2d, ex3, hfeat, zeros_c)


def kernel(x, edge_index, batch, W1, att_src1, att_dst1, b1,
           W2, att_src2, att_dst2, b2, fc1_w, fc1_b, fc2_w, fc2_b):
    n, din = x.shape
    e = edge_index.shape[1]
    c1 = att_src1.shape[2]
    c2 = att_src2.shape[2]
    er = e // ROWE

    src = edge_index[0].astype(jnp.int32).reshape(er, ROWE)
    dst = edge_index[1].astype(jnp.int32).reshape(er, ROWE)
    batch2d = batch.astype(jnp.int32).reshape(1, n)

    w1r = W1.reshape(din, H, c1)
    ws1 = jnp.einsum('khc,hc->kh', w1r, att_src1[0])
    wd1 = jnp.einsum('khc,hc->kh', w1r, att_dst1[0])
    w1h = W1.reshape(din, 2 * H, c1 // 2).transpose(1, 0, 2)
    w2r = W2.reshape(H * c1, H, c2)
    ws2 = jnp.einsum('khc,hc->kh', w2r, att_src2[0])
    wd2 = jnp.einsum('khc,hc->kh', w2r, att_dst2[0])
    w2h = w2r.transpose(1, 0, 2)

    zeros8 = jnp.zeros((n, H), jnp.float32)
    zerosw = jnp.zeros((n, 64), jnp.float32)

    a1s, a1d, gmax1 = _attn_scalars(x, ws1, wd1, nb=1000)
    h1 = _matmul_heads(x, w1h, nb=1000)
    gm1 = jnp.concatenate([gmax1.reshape(H), jnp.zeros((8,), jnp.float32)])
    ex1, den1 = _edge_sc(src, dst, a1s, a1d, gm1, zeros8)
    agg1 = _agg_sc(src, dst, ex1, h1, zerosw, qblk=2, vh=2 * H, shift=1)
    z1 = _norm(agg1.reshape(H, 2, n, c1 // 2), den1, b1.reshape(1, H * c1), nb=1000)

    a2s, a2d, gmax2 = _attn_scalars(z1, ws2, wd2, nb=1000)
    h2 = _matmul_heads(z1, w2h, nb=1000)
    gm2 = jnp.concatenate([gmax2.reshape(H), jnp.zeros((8,), jnp.float32)])
    ex2, den2 = _edge_sc(src, dst, a2s, a2d, gm2, zeros8)
    agg2 = _agg_sc(src, dst, ex2, h2, zerosw, qblk=2, vh=H, shift=0)
    z2 = _norm(agg2.reshape(H, 1, n, c2), den2, b2.reshape(1, H * c2), nb=1000)

    return _pool_mlp(z2, batch2d, fc1_w, fc1_b, fc2_w, fc2_b)


# speedup vs baseline: 1.0430x; 1.0430x over previous
import functools

import jax
import jax.numpy as jnp
from jax import lax
from jax.experimental import pallas as pl
from jax.experimental.pallas import tpu as pltpu
from jax.experimental.pallas import tpu_sc as plsc

H = 8
NEG_SLOPE = 0.2
NUM_GRAPHS = 64
NSC = 2
NTEC = 16
ROWE = 128


def _lrelu(t):
    return jnp.where(t >= 0, t, NEG_SLOPE * t)


def _elu(t):
    return jnp.where(t > 0, t, jnp.exp(t) - 1.0)


def _attn_body(x_ref, ws_ref, wd_ref, as_ref, ad_ref, gmax_ref, acc_ref):
    i = pl.program_id(0)
    a_s = jnp.dot(x_ref[...], ws_ref[...], preferred_element_type=jnp.float32)
    a_d = jnp.dot(x_ref[...], wd_ref[...], preferred_element_type=jnp.float32)
    as_ref[...] = a_s
    ad_ref[...] = a_d
    bmax = jnp.max(a_s, axis=0, keepdims=True)

    @pl.when(i == 0)
    def _():
        acc_ref[...] = bmax

    @pl.when(i > 0)
    def _():
        acc_ref[...] = jnp.maximum(acc_ref[...], bmax)

    @pl.when(i == pl.num_programs(0) - 1)
    def _():
        gmax_ref[...] = acc_ref[...]


def _attn_scalars(x, ws, wd, nb):
    n, din = x.shape
    grid = (n // nb,)
    return pl.pallas_call(
        _attn_body,
        grid=grid,
        in_specs=[
            pl.BlockSpec((nb, din), lambda i: (i, 0)),
            pl.BlockSpec((din, H), lambda i: (0, 0)),
            pl.BlockSpec((din, H), lambda i: (0, 0)),
        ],
        out_specs=[
            pl.BlockSpec((nb, H), lambda i: (i, 0)),
            pl.BlockSpec((nb, H), lambda i: (i, 0)),
            pl.BlockSpec((1, H), lambda i: (0, 0)),
        ],
        out_shape=[
            jax.ShapeDtypeStruct((n, H), jnp.float32),
            jax.ShapeDtypeStruct((n, H), jnp.float32),
            jax.ShapeDtypeStruct((1, H), jnp.float32),
        ],
        scratch_shapes=[pltpu.VMEM((1, H), jnp.float32)],
    )(x, ws, wd)


def _mm_body(x_ref, w_ref, out_ref):
    out_ref[...] = jnp.dot(x_ref[...], w_ref[0],
                           preferred_element_type=jnp.float32)


def _matmul_heads(x, w_heads, nb):
    n, k = x.shape
    nh, _, c = w_heads.shape
    nblk = n // nb
    return pl.pallas_call(
        _mm_body,
        grid=(nblk, nh),
        in_specs=[
            pl.BlockSpec((nb, k), lambda i, h: (i, 0)),
            pl.BlockSpec((1, k, c), lambda i, h: (h, 0, 0)),
        ],
        out_specs=pl.BlockSpec((nb, c), lambda i, h: (h * nblk + i, 0)),
        out_shape=jax.ShapeDtypeStruct((nh * n, c), jnp.float32),
    )(x, w_heads)


def _norm_body(c, sp, agg_ref, den_ref, b_ref, out_ref):
    den = den_ref[0] + den_ref[1]
    cw = c // sp
    for h in range(H):
        d = jnp.clip(den[:, h:h + 1], 1e-16, None)
        for t in range(sp):
            v = agg_ref[h, t] / d + b_ref[0, h * c + t * cw:h * c + (t + 1) * cw]
            out_ref[:, h * c + t * cw:h * c + (t + 1) * cw] = _elu(v)


def _norm(agg, den, bias, nb):
    _, sp, n, cw = agg.shape
    c = sp * cw
    return pl.pallas_call(
        functools.partial(_norm_body, c, sp),
        grid=(n // nb,),
        in_specs=[
            pl.BlockSpec((H, sp, nb, cw), lambda i: (0, 0, i, 0)),
            pl.BlockSpec((2, nb, H), lambda i: (0, i, 0)),
            pl.BlockSpec((1, H * c), lambda i: (0, 0)),
        ],
        out_specs=pl.BlockSpec((nb, H * c), lambda i: (i, 0)),
        out_shape=jax.ShapeDtypeStruct((n, H * c), jnp.float32),
    )(agg, den, bias)


def _pool_body(z_ref, b_ref, w1_ref, b1_ref, w2_ref, b2_ref, out_ref):
    n = z_ref.shape[0]
    oh = (b_ref[...] == lax.broadcasted_iota(jnp.int32, (NUM_GRAPHS, n), 0))
    oh = oh.astype(jnp.float32)
    s = jnp.dot(oh, z_ref[...], preferred_element_type=jnp.float32)
    cnt = jnp.sum(oh, axis=1, keepdims=True)
    g = s / jnp.clip(cnt, 1.0, None)
    t = _elu(jnp.dot(g, w1_ref[...], preferred_element_type=jnp.float32)
             + b1_ref[...])
    out_ref[...] = jnp.dot(t, w2_ref[...],
                           preferred_element_type=jnp.float32) + b2_ref[...]


def _pool_mlp(z, batch2d, fc1_w, fc1_b, fc2_w, fc2_b):
    n, d = z.shape
    dh, do = fc1_w.shape[1], fc2_w.shape[1]
    return pl.pallas_call(
        _pool_body,
        out_shape=jax.ShapeDtypeStruct((NUM_GRAPHS, do), jnp.float32),
    )(z, batch2d, fc1_w, fc1_b.reshape(1, dh), fc2_w, fc2_b.reshape(1, do))


def _edge_sc(src2d, dst2d, ts, td, gmax, zeros8):
    er = src2d.shape[0]
    n = ts.shape[0]
    nrows = n // NTEC
    base = er // (NSC * NTEC)
    rem = er - base * NSC * NTEC
    nbody = base // 2
    assert base == 2 * nbody

    mesh = plsc.VectorSubcoreMesh(core_axis_name="c", subcore_axis_name="s")

    @functools.partial(
        pl.kernel,
        mesh=mesh,
        compiler_params=pltpu.CompilerParams(use_tc_tiling_on_sc=False, needs_layout_passes=False),
        out_type=[
            jax.ShapeDtypeStruct((er, ROWE, H), jnp.float32),
            jax.ShapeDtypeStruct((NSC, n, H), jnp.float32),
        ],
        scratch_types=[
            pltpu.VMEM((2, ROWE), jnp.int32),
            pltpu.VMEM((2, ROWE), jnp.int32),
            pltpu.VMEM((2, ROWE), jnp.int32),
            pltpu.VMEM((2, ROWE, H), jnp.float32),
            pltpu.VMEM((2, ROWE, H), jnp.float32),
            pltpu.VMEM((2, ROWE, H), jnp.float32),
            pltpu.VMEM((16,), jnp.float32),
            pltpu.VMEM_SHARED((n, H), jnp.float32),
            pltpu.SemaphoreType.DMA,
            pltpu.SemaphoreType.DMA,
            pltpu.SemaphoreType.DMA,
            pltpu.SemaphoreType.DMA,
            pltpu.SemaphoreType.DMA,
            pltpu.SemaphoreType.DMA,
            pltpu.SemaphoreType.DMA,
            pltpu.SemaphoreType.DMA,
        ],
    )
    def k(src_h, dst_h, ts_h, td_h, gmax_h, z8_h, ex_h, den_h,
          srcv, dstv, dsts, srows, drows, exrows, gbuf, denacc,
          semi0, semi1, semr0, semr1, semw0, semw1, semd0, semd1):
        c = lax.axis_index("c")
        s = lax.axis_index("s")
        wid = c * NTEC + s
        semi = [semi0, semi1]
        semr = [semr0, semr1]
        semw = [semw0, semw1]
        semd = [semd0, semd1]
        pltpu.sync_copy(z8_h.at[pl.ds(s * nrows, nrows), :],
                        denacc.at[pl.ds(s * nrows, nrows), :])
        pltpu.sync_copy(gmax_h, gbuf)
        plsc.subcore_barrier()

        lane = lax.broadcasted_iota(jnp.int32, (16,), 0)
        ci = lane & 7
        gv = plsc.load_gather(gbuf, [ci])
        start = wid * base + jnp.minimum(wid, rem)
        extra = wid < rem

        def fire_idx(k_, r):
            pltpu.async_copy(src_h.at[r], srcv.at[k_], semi[k_])
            pltpu.async_copy(dst_h.at[r], dstv.at[k_], semi[k_])

        def wait_idx(k_):
            pltpu.make_async_copy(src_h.at[0], srcv.at[k_], semi[k_]).wait()
            pltpu.make_async_copy(dst_h.at[0], dstv.at[k_], semi[k_]).wait()

        def prep_fire_rows(k_):
            def sb(i, _):
                dsts[k_, pl.ds(i * 16, 16)] = dstv[k_, pl.ds(i * 16, 16)]
                return 0
            lax.fori_loop(0, ROWE // 16, sb, 0)
            pltpu.async_copy(ts_h.at[srcv.at[k_]], srows.at[k_], semr[k_])
            pltpu.async_copy(td_h.at[dstv.at[k_]], drows.at[k_], semr[k_])

        def wait_rows(k_):
            pltpu.make_async_copy(ts_h.at[srcv.at[k_]], srows.at[k_],
                                  semr[k_]).wait()
            pltpu.make_async_copy(td_h.at[dstv.at[k_]], drows.at[k_],
                                  semr[k_]).wait()

        def compute(k_):
            ksp = jnp.full((16,), k_, jnp.int32)

            def body(i, _):
                ri = i * 2 + (lane >> 3)
                sv = plsc.load_gather(srows, [ksp, ri, ci])
                dv = plsc.load_gather(drows, [ksp, ri, ci])
                ex = jnp.exp(_lrelu(sv + dv) - _lrelu(gv + dv))
                plsc.store_scatter(exrows, [ksp, ri, ci], ex)
                return 0

            lax.fori_loop(0, ROWE * H // 16, body, 0)

        def fire_writes(k_, r):
            pltpu.async_copy(exrows.at[k_], ex_h.at[r], semw[k_])
            pltpu.async_copy(exrows.at[k_], denacc.at[dsts.at[k_]],
                             semd[k_], add=True)

        def wait_writes(k_):
            pltpu.make_async_copy(exrows.at[k_], ex_h.at[0], semw[k_]).wait()
            pltpu.make_async_copy(exrows.at[k_], denacc.at[dsts.at[k_]],
                                  semd[k_]).wait()

        fire_idx(0, start)

        def blk(g, _):
            wait_idx(0)

            @pl.when(g > 0)
            def _():
                wait_writes(0)

            prep_fire_rows(0)

            @pl.when(g > 0)
            def _():
                wait_rows(1)
                compute(1)
                fire_writes(1, start + 2 * g - 1)

            fire_idx(1, start + 2 * g + 1)
            wait_idx(1)

            @pl.when(g > 0)
            def _():
                wait_writes(1)

            prep_fire_rows(1)
            wait_rows(0)
            compute(0)
            fire_writes(0, start + 2 * g)

            @pl.when(g < nbody - 1)
            def _():
                fire_idx(0, start + 2 * g + 2)

            return 0

        lax.fori_loop(0, nbody, blk, 0)
        wait_rows(1)
        compute(1)
        pltpu.sync_copy(exrows.at[1], ex_h.at[start + base - 1])
        pltpu.sync_copy(exrows.at[1], denacc.at[dsts.at[1]], add=True)
        wait_writes(0)

        @pl.when(extra)
        def _():
            r = start + base
            pltpu.sync_copy(src_h.at[r], srcv.at[0])
            pltpu.sync_copy(dst_h.at[r], dstv.at[0])

            def sb(i, _):
                dsts[0, pl.ds(i * 16, 16)] = dstv[0, pl.ds(i * 16, 16)]
                return 0
            lax.fori_loop(0, ROWE // 16, sb, 0)
            pltpu.async_copy(ts_h.at[srcv.at[0]], srows.at[0], semr0)
            pltpu.async_copy(td_h.at[dstv.at[0]], drows.at[0], semr0)
            wait_rows(0)
            compute(0)
            pltpu.sync_copy(exrows.at[0], ex_h.at[r])
            pltpu.sync_copy(exrows.at[0], denacc.at[dsts.at[0]], add=True)

        plsc.subcore_barrier()
        pltpu.sync_copy(denacc.at[pl.ds(s * nrows, nrows), :],
                        den_h.at[c, pl.ds(s * nrows, nrows), :])

    return k(src2d, dst2d, ts, td, gmax, zeros8)


def _agg_sc(src2d, dst2d, ex3, hfeat, zeros_c, qblk, vh, shift):
    er = src2d.shape[0]
    n = zeros_c.shape[0]
    cw = zeros_c.shape[1]
    nrows = n // NTEC
    hpc = vh // NSC
    base = er // NTEC
    nblk = base // qblk
    nbody = nblk // 3
    rem = er - base * NTEC
    assert base == nblk * qblk and nblk == 3 * nbody

    mesh = plsc.VectorSubcoreMesh(core_axis_name="c", subcore_axis_name="s")

    @functools.partial(
        pl.kernel,
        mesh=mesh,
        compiler_params=pltpu.CompilerParams(use_tc_tiling_on_sc=False, needs_layout_passes=False),
        out_type=jax.ShapeDtypeStruct((vh * n, cw), jnp.float32),
        scratch_types=[
            pltpu.VMEM((3, qblk, ROWE), jnp.int32),
            pltpu.VMEM((3, qblk, ROWE), jnp.int32),
            pltpu.VMEM((3, qblk, ROWE), jnp.int32),
            pltpu.VMEM((3, qblk, ROWE), jnp.int32),
            pltpu.VMEM((3, qblk, ROWE, H), jnp.float32),
            pltpu.VMEM((3, qblk, ROWE, cw), jnp.float32),
            pltpu.VMEM_SHARED((n, cw), jnp.float32),
            pltpu.SemaphoreType.DMA,
            pltpu.SemaphoreType.DMA,
            pltpu.SemaphoreType.DMA,
            pltpu.SemaphoreType.DMA,
            pltpu.SemaphoreType.DMA,
            pltpu.SemaphoreType.DMA,
            pltpu.SemaphoreType.DMA,
            pltpu.SemaphoreType.DMA,
            pltpu.SemaphoreType.DMA,
        ],
    )
    def k(src_h, dst_h, ex_h, hf_h, zc_h, out_h,
          srcv, dstv, dsts, gidx, exv, gbuf, uacc,
          semi0, semi1, semi2, semg0, semg1, semg2, sems0, sems1, sems2):
        c = lax.axis_index("c")
        s = lax.axis_index("s")
        start = s * base + jnp.minimum(s, rem)
        extra = s < rem
        semi = [semi0, semi1, semi2]
        semg = [semg0, semg1, semg2]
        sems = [sems0, sems1, sems2]

        def fire_idx(k_, row0, q=qblk):
            pltpu.async_copy(src_h.at[pl.ds(row0, q)], srcv.at[k_, pl.ds(0, q)], semi[k_])
            pltpu.async_copy(dst_h.at[pl.ds(row0, q)], dstv.at[k_, pl.ds(0, q)], semi[k_])
            pltpu.async_copy(ex_h.at[pl.ds(row0, q)], exv.at[k_, pl.ds(0, q)], semi[k_])

        def wait_idx(k_, q=qblk):
            pltpu.make_async_copy(src_h.at[pl.ds(0, q)], srcv.at[k_, pl.ds(0, q)], semi[k_]).wait()
            pltpu.make_async_copy(dst_h.at[pl.ds(0, q)], dstv.at[k_, pl.ds(0, q)], semi[k_]).wait()
            pltpu.make_async_copy(ex_h.at[pl.ds(0, q)], exv.at[k_, pl.ds(0, q)], semi[k_]).wait()

        def fire_gather(k_, v, q=qblk):
            def rbody(i, _):
                qq = i >> 3
                ii = i & 7
                gidx[k_, qq, pl.ds(ii * 16, 16)] = (
                    srcv[k_, qq, pl.ds(ii * 16, 16)] + v * n)
                dsts[k_, qq, pl.ds(ii * 16, 16)] = dstv[k_, qq, pl.ds(ii * 16, 16)]
                return 0
            lax.fori_loop(0, q * 8, rbody, 0)
            for qq in range(q):
                pltpu.async_copy(hf_h.at[gidx.at[k_, qq]], gbuf.at[k_, qq],
                                 semg[k_])

        def wait_gather(k_, q=qblk):
            for qq in range(q):
                pltpu.make_async_copy(hf_h.at[gidx.at[k_, qq]],
                                      gbuf.at[k_, qq], semg[k_]).wait()

        def scale_scatter(k_, v, q=qblk):
            hsp = jnp.full((16,), 0, jnp.int32) + (v >> shift)
            ksp = jnp.full((16,), k_, jnp.int32)

            def row(r, _):
                qq = r >> 7
                rr = r & 127
                rsp = jnp.full((16,), 0, jnp.int32) + rr
                qsp = jnp.full((16,), 0, jnp.int32) + qq
                exb = plsc.load_gather(exv, [ksp, qsp, rsp, hsp])
                for cc in range(cw // 16):
                    gbuf[k_, qq, rr, pl.ds(cc * 16, 16)] = (
                        gbuf[k_, qq, rr, pl.ds(cc * 16, 16)] * exb)
                return 0

            lax.fori_loop(0, q * ROWE, row, 0)
            for qq in range(q):
                pltpu.async_copy(gbuf.at[k_, qq], uacc.at[dsts.at[k_, qq]],
                                 sems[k_], add=True)

        def wait_scatter(k_, q=qblk):
            for qq in range(q):
                pltpu.make_async_copy(gbuf.at[k_, qq],
                                      uacc.at[dsts.at[k_, qq]],
                                      sems[k_]).wait()

        for j in range(hpc):
            v = c * hpc + j
            pltpu.sync_copy(zc_h.at[pl.ds(s * nrows, nrows), :],
                            uacc.at[pl.ds(s * nrows, nrows), :])
            plsc.subcore_barrier()

            fire_idx(0, start)
            fire_idx(1, start + qblk)
            fire_idx(2, start + 2 * qblk)

            def blk(g, _):
                wait_idx(0)

                @pl.when(g > 0)
                def _():
                    wait_scatter(0)

                fire_gather(0, v)

                @pl.when(g > 0)
                def _():
                    wait_gather(2)
                    scale_scatter(2, v)
                    fire_idx(2, start + (3 * g + 2) * qblk)

                wait_idx(1)

                @pl.when(g > 0)
                def _():
                    wait_scatter(1)

                fire_gather(1, v)
                wait_gather(0)
                scale_scatter(0, v)

                @pl.when(g < nbody - 1)
                def _():
                    fire_idx(0, start + (3 * g + 3) * qblk)

                wait_idx(2)

                @pl.when(g > 0)
                def _():
                    wait_scatter(2)

                fire_gather(2, v)
                wait_gather(1)
                scale_scatter(1, v)

                @pl.when(g < nbody - 1)
                def _():
                    fire_idx(1, start + (3 * g + 4) * qblk)

                return 0

            lax.fori_loop(0, nbody, blk, 0)
            wait_gather(2)
            scale_scatter(2, v)
            wait_scatter(0)
            wait_scatter(1)
            wait_scatter(2)

            @pl.when(extra)
            def _():
                fire_idx(0, start + base, 1)
                wait_idx(0, 1)
                fire_gather(0, v, 1)
                wait_gather(0, 1)
                scale_scatter(0, v, 1)
                wait_scatter(0, 1)

            plsc.subcore_barrier()
            pltpu.sync_copy(uacc.at[pl.ds(s * nrows, nrows), :],
                            out_h.at[pl.ds(v * n + s * nrows, nrows), :])
            plsc.subcore_barrier()

    return k(src2d, dst2d, ex3, hfeat, zeros_c)


def kernel(x, edge_index, batch, W1, att_src1, att_dst1, b1,
           W2, att_src2, att_dst2, b2, fc1_w, fc1_b, fc2_w, fc2_b):
    n, din = x.shape
    e = edge_index.shape[1]
    c1 = att_src1.shape[2]
    c2 = att_src2.shape[2]
    er = e // ROWE

    src = edge_index[0].astype(jnp.int32).reshape(er, ROWE)
    dst = edge_index[1].astype(jnp.int32).reshape(er, ROWE)
    batch2d = batch.astype(jnp.int32).reshape(1, n)

    w1r = W1.reshape(din, H, c1)
    ws1 = jnp.einsum('khc,hc->kh', w1r, att_src1[0])
    wd1 = jnp.einsum('khc,hc->kh', w1r, att_dst1[0])
    w1h = W1.reshape(din, 2 * H, c1 // 2).transpose(1, 0, 2)
    w2r = W2.reshape(H * c1, H, c2)
    ws2 = jnp.einsum('khc,hc->kh', w2r, att_src2[0])
    wd2 = jnp.einsum('khc,hc->kh', w2r, att_dst2[0])
    w2h = w2r.transpose(1, 0, 2)

    zeros8 = jnp.zeros((n, H), jnp.float32)
    zerosw = jnp.zeros((n, 64), jnp.float32)

    a1s, a1d, gmax1 = _attn_scalars(x, ws1, wd1, nb=1000)
    h1 = _matmul_heads(x, w1h, nb=1000)
    gm1 = jnp.concatenate([gmax1.reshape(H), jnp.zeros((8,), jnp.float32)])
    ex1, den1 = _edge_sc(src, dst, a1s, a1d, gm1, zeros8)
    agg1 = _agg_sc(src, dst, ex1, h1, zerosw, qblk=2, vh=2 * H, shift=1)
    z1 = _norm(agg1.reshape(H, 2, n, c1 // 2), den1, b1.reshape(1, H * c1), nb=1000)

    a2s, a2d, gmax2 = _attn_scalars(z1, ws2, wd2, nb=1000)
    h2 = _matmul_heads(z1, w2h, nb=1000)
    gm2 = jnp.concatenate([gmax2.reshape(H), jnp.zeros((8,), jnp.float32)])
    ex2, den2 = _edge_sc(src, dst, a2s, a2d, gm2, zeros8)
    agg2 = _agg_sc(src, dst, ex2, h2, zerosw, qblk=2, vh=H, shift=0)
    z2 = _norm(agg2.reshape(H, 1, n, c2), den2, b2.reshape(1, H * c2), nb=1000)

    return _pool_mlp(z2, batch2d, fc1_w, fc1_b, fc2_w, fc2_b)
